# baseline (speedup 1.0000x reference)
import functools

import jax
import jax.numpy as jnp
from jax import lax
from jax.experimental import pallas as pl
from jax.experimental.pallas import tpu as pltpu
from jax.experimental.pallas import tpu_sc as plsc

B = 4096
NF = 128
H1 = 1024
H2 = 512
BT = 256


@functools.lru_cache(maxsize=1)
def _make_gather6():
    info = plsc.get_sparse_core_info()
    nc, ns = info.num_cores, info.num_subcores
    nw = nc * ns
    bpw = B // nw

    mesh = plsc.VectorSubcoreMesh(core_axis_name="c", subcore_axis_name="s")

    @functools.partial(
        pl.kernel,
        mesh=mesh,
        out_type=[jax.ShapeDtypeStruct((B, NF), jnp.float32) for _ in range(6)],
        scratch_types=[
            pltpu.VMEM((bpw,), jnp.int32),
            pltpu.VMEM((bpw, NF), jnp.float32),
            pltpu.SemaphoreType.DMA,
        ],
    )
    def gather6(ut, nt, gt, agt, ot, adt,
                ui, ni, gi, ai, oi, adi,
                xu, xn, xg, xa, xo, xad,
                idx_v, rows_v, sem):
        wid = lax.axis_index("s") * nc + lax.axis_index("c")
        base = wid * bpw
        for tab, idx, out in ((ut, ui, xu), (nt, ni, xn), (gt, gi, xg),
                              (agt, ai, xa), (ot, oi, xo), (adt, adi, xad)):
            pltpu.sync_copy(idx.at[pl.ds(base, bpw)], idx_v)
            pltpu.async_copy(tab.at[idx_v], rows_v, sem).wait()
            pltpu.sync_copy(rows_v, out.at[pl.ds(base, bpw)])

    return gather6


def _mlp_body(xu, xn, xg, xa, xo, xad, w1, b1, w2, b2, wfct, scal, out):
    x = jnp.concatenate(
        [xu[...], xn[...], xg[...], xa[...], xo[...], xad[...]], axis=1)
    h = lax.dot_general(x, w1[...], (((1,), (0,)), ((), ())),
                        preferred_element_type=jnp.float32)
    h = jnp.maximum(h + b1[...], 0.0)
    h = lax.dot_general(h, w2[...], (((1,), (0,)), ((), ())),
                        preferred_element_type=jnp.float32)
    h = jnp.maximum(h + b2[...], 0.0)
    z = jnp.sum(h * wfct[...], axis=1, keepdims=True) + scal[0, 0]
    sig = 1.0 / (1.0 + jnp.exp(-z))
    mn, mx = scal[0, 1], scal[0, 2]
    out[...] = sig * (mx - mn) + mn


def _mlp(xu, xn, xg, xa, xo, xad, W1, b1, W2, b2, WfcT, scal):
    grid = (B // BT,)
    emb_spec = pl.BlockSpec((BT, NF), lambda i: (i, 0))
    full = lambda shape: pl.BlockSpec(shape, lambda i: (0,) * len(shape))
    return pl.pallas_call(
        _mlp_body,
        grid=grid,
        in_specs=[
            emb_spec, emb_spec, emb_spec, emb_spec, emb_spec, emb_spec,
            full((6 * NF, H1)),
            full((1, H1)),
            full((H1, H2)),
            full((1, H2)),
            full((1, H2)),
            pl.BlockSpec(memory_space=pltpu.SMEM),
        ],
        out_specs=pl.BlockSpec((BT, 1), lambda i: (i, 0)),
        out_shape=jax.ShapeDtypeStruct((B, 1), jnp.float32),
        compiler_params=pltpu.CompilerParams(
            dimension_semantics=("arbitrary",)),
    )(xu, xn, xg, xa, xo, xad, W1, b1, W2, b2, WfcT, scal)


def kernel(users, news, gender, age, occupation, address,
           user_table, news_table, gender_table, age_table,
           occupation_table, address_table,
           W1, b1, W2, b2, Wfc, bfc,
           min_rating=0.5, max_rating=5.0):
    xu, xn, xg, xa, xo, xad = _make_gather6()(
        user_table, news_table, gender_table, age_table,
        occupation_table, address_table,
        users.astype(jnp.int32), news.astype(jnp.int32),
        gender.astype(jnp.int32), age.astype(jnp.int32),
        occupation.astype(jnp.int32), address.astype(jnp.int32))
    scal = jnp.stack([
        jnp.asarray(bfc, jnp.float32).reshape(()),
        jnp.asarray(min_rating, jnp.float32).reshape(()),
        jnp.asarray(max_rating, jnp.float32).reshape(()),
    ]).reshape(1, 3)
    return _mlp(xu, xn, xg, xa, xo, xad,
                W1, b1.reshape(1, H1), W2, b2.reshape(1, H2),
                Wfc.reshape(1, H2), scal)


# speedup vs baseline: 3.1640x; 3.1640x over previous
import functools

import jax
import jax.numpy as jnp
from jax import lax
from jax.experimental import pallas as pl
from jax.experimental.pallas import tpu as pltpu
from jax.experimental.pallas import tpu_sc as plsc

B = 4096
NF = 128
H1 = 1024
H2 = 512
BT = 256
NS_PAD = 32


@functools.lru_cache(maxsize=1)
def _make_gather2():
    info = plsc.get_sparse_core_info()
    nc, ns = info.num_cores, info.num_subcores
    nw = nc * ns
    bpw = B // nw

    mesh = plsc.VectorSubcoreMesh(core_axis_name="c", subcore_axis_name="s")

    @functools.partial(
        pl.kernel,
        mesh=mesh,
        out_type=[jax.ShapeDtypeStruct((B, NF), jnp.float32) for _ in range(2)],
        scratch_types=[
            pltpu.VMEM((bpw,), jnp.int32),
            pltpu.VMEM((bpw,), jnp.int32),
            pltpu.VMEM((bpw, NF), jnp.float32),
            pltpu.VMEM((bpw, NF), jnp.float32),
            pltpu.SemaphoreType.DMA,
            pltpu.SemaphoreType.DMA,
            pltpu.SemaphoreType.DMA,
        ],
    )
    def gather2(ut, nt, ui, ni, xu, xn,
                idx_u, idx_n, rows_u, rows_n, sem_u, sem_n, sem_w):
        wid = lax.axis_index("s") * nc + lax.axis_index("c")
        base = wid * bpw
        pltpu.sync_copy(ui.at[pl.ds(base, bpw)], idx_u)
        pltpu.sync_copy(ni.at[pl.ds(base, bpw)], idx_n)
        cu = pltpu.async_copy(ut.at[idx_u], rows_u, sem_u)
        cn = pltpu.async_copy(nt.at[idx_n], rows_n, sem_n)
        cu.wait()
        wu = pltpu.async_copy(rows_u, xu.at[pl.ds(base, bpw)], sem_w)
        cn.wait()
        wn = pltpu.async_copy(rows_n, xn.at[pl.ds(base, bpw)], sem_w)
        wu.wait()
        wn.wait()

    return gather2


def _mlp_body(xu, xn, gi, ai, oi, adi, w1, spad, b1, w2, b2, wfct, scal,
              out, p):
    @pl.when(pl.program_id(0) == 0)
    def _():
        p[...] = lax.dot_general(spad[...], w1[2 * NF:6 * NF, :],
                                 (((1,), (0,)), ((), ())),
                                 preferred_element_type=jnp.float32)

    x2 = jnp.concatenate([xu[...], xn[...]], axis=1)
    h = lax.dot_general(x2, w1[0:2 * NF, :], (((1,), (0,)), ((), ())),
                        preferred_element_type=jnp.float32)
    lanes = lax.broadcasted_iota(jnp.int32, (BT, NS_PAD), 1)
    oh = ((lanes == gi[0]) | (lanes == (ai[0] + 2)) |
          (lanes == (oi[0] + 7)) | (lanes == (adi[0] + 17)))
    h = h + lax.dot_general(oh.astype(jnp.float32), p[...],
                            (((1,), (0,)), ((), ())),
                            preferred_element_type=jnp.float32)
    h = jnp.maximum(h + b1[...], 0.0)
    h = lax.dot_general(h, w2[...], (((1,), (0,)), ((), ())),
                        preferred_element_type=jnp.float32)
    h = jnp.maximum(h + b2[...], 0.0)
    z = jnp.sum(h * wfct[...], axis=1, keepdims=True) + scal[0, 0]
    sig = 1.0 / (1.0 + jnp.exp(-z))
    mn, mx = scal[0, 1], scal[0, 2]
    out[...] = sig * (mx - mn) + mn


def _mlp(xu, xn, gi3, ai3, oi3, adi3, W1, spad, b1, W2, b2, WfcT, scal):
    grid = (B // BT,)
    emb_spec = pl.BlockSpec((BT, NF), lambda i: (i, 0))
    idx_spec = pl.BlockSpec((1, BT, 1), lambda i: (i, 0, 0))
    full = lambda shape: pl.BlockSpec(shape, lambda i: (0,) * len(shape))
    return pl.pallas_call(
        _mlp_body,
        grid=grid,
        in_specs=[
            emb_spec, emb_spec,
            idx_spec, idx_spec, idx_spec, idx_spec,
            full((6 * NF, H1)),
            full((NS_PAD, 4 * NF)),
            full((1, H1)),
            full((H1, H2)),
            full((1, H2)),
            full((1, H2)),
            pl.BlockSpec(memory_space=pltpu.SMEM),
        ],
        out_specs=pl.BlockSpec((BT, 1), lambda i: (i, 0)),
        out_shape=jax.ShapeDtypeStruct((B, 1), jnp.float32),
        scratch_shapes=[pltpu.VMEM((NS_PAD, H1), jnp.float32)],
        compiler_params=pltpu.CompilerParams(
            dimension_semantics=("arbitrary",)),
    )(xu, xn, gi3, ai3, oi3, adi3, W1, spad, b1, W2, b2, WfcT, scal)


def kernel(users, news, gender, age, occupation, address,
           user_table, news_table, gender_table, age_table,
           occupation_table, address_table,
           W1, b1, W2, b2, Wfc, bfc,
           min_rating=0.5, max_rating=5.0):
    xu, xn = _make_gather2()(
        user_table, news_table,
        users.astype(jnp.int32), news.astype(jnp.int32))
    spad = jnp.zeros((NS_PAD, 4 * NF), jnp.float32)
    spad = spad.at[0:2, 0:NF].set(gender_table)
    spad = spad.at[2:7, NF:2 * NF].set(age_table)
    spad = spad.at[7:17, 2 * NF:3 * NF].set(occupation_table)
    spad = spad.at[17:23, 3 * NF:4 * NF].set(address_table)
    scal = jnp.stack([
        jnp.asarray(bfc, jnp.float32).reshape(()),
        jnp.asarray(min_rating, jnp.float32).reshape(()),
        jnp.asarray(max_rating, jnp.float32).reshape(()),
    ]).reshape(1, 3)
    shape3 = (B // BT, BT, 1)
    return _mlp(xu, xn,
                gender.astype(jnp.int32).reshape(shape3),
                age.astype(jnp.int32).reshape(shape3),
                occupation.astype(jnp.int32).reshape(shape3),
                address.astype(jnp.int32).reshape(shape3),
                W1, spad, b1.reshape(1, H1), W2, b2.reshape(1, H2),
                Wfc.reshape(1, H2), scal)


# speedup vs baseline: 3.8407x; 1.2139x over previous
import functools

import jax
import jax.numpy as jnp
from jax import lax
from jax.experimental import pallas as pl
from jax.experimental.pallas import tpu as pltpu
from jax.experimental.pallas import tpu_sc as plsc

B = 4096
NF = 128
H1 = 1024
H2 = 512
BT = 512
NS_PAD = 32
NCHUNK = 2


@functools.lru_cache(maxsize=None)
def _make_gather2(n):
    info = plsc.get_sparse_core_info()
    nc, ns = info.num_cores, info.num_subcores
    nw = nc * ns
    bpw = n // nw

    mesh = plsc.VectorSubcoreMesh(core_axis_name="c", subcore_axis_name="s")

    @functools.partial(
        pl.kernel,
        mesh=mesh,
        out_type=[jax.ShapeDtypeStruct((n, NF), jnp.float32) for _ in range(2)],
        scratch_types=[
            pltpu.VMEM((bpw,), jnp.int32),
            pltpu.VMEM((bpw,), jnp.int32),
            pltpu.VMEM((bpw, NF), jnp.float32),
            pltpu.VMEM((bpw, NF), jnp.float32),
            pltpu.SemaphoreType.DMA,
            pltpu.SemaphoreType.DMA,
            pltpu.SemaphoreType.DMA,
        ],
    )
    def gather2(ut, nt, ui, ni, xu, xn,
                idx_u, idx_n, rows_u, rows_n, sem_u, sem_n, sem_w):
        wid = lax.axis_index("s") * nc + lax.axis_index("c")
        base = wid * bpw
        iu = pltpu.async_copy(ui.at[pl.ds(base, bpw)], idx_u, sem_u)
        inn = pltpu.async_copy(ni.at[pl.ds(base, bpw)], idx_n, sem_n)
        iu.wait()
        inn.wait()
        cu = pltpu.async_copy(ut.at[idx_u], rows_u, sem_u)
        cn = pltpu.async_copy(nt.at[idx_n], rows_n, sem_n)
        cu.wait()
        wu = pltpu.async_copy(rows_u, xu.at[pl.ds(base, bpw)], sem_w)
        cn.wait()
        wn = pltpu.async_copy(rows_n, xn.at[pl.ds(base, bpw)], sem_w)
        wu.wait()
        wn.wait()

    return gather2


def _mlp_body(xu, xn, idx4, w1, spad, b1, w2, b2, wfct, scal, out, p):
    @pl.when(pl.program_id(0) == 0)
    def _():
        p[...] = lax.dot_general(spad[...], w1[2 * NF:6 * NF, :],
                                 (((1,), (0,)), ((), ())),
                                 preferred_element_type=jnp.float32
                                 ).astype(jnp.bfloat16)

    x2 = jnp.concatenate([xu[...], xn[...]], axis=1).astype(jnp.bfloat16)
    h = lax.dot_general(x2, w1[0:2 * NF, :], (((1,), (0,)), ((), ())),
                        preferred_element_type=jnp.float32)
    rows = lax.broadcasted_iota(jnp.int32, (NS_PAD, BT), 0)
    oht = ((rows == idx4[0:1, :]) | (rows == (idx4[1:2, :] + 2)) |
           (rows == (idx4[2:3, :] + 7)) | (rows == (idx4[3:4, :] + 17)))
    h = h + lax.dot_general(oht.astype(jnp.bfloat16), p[...],
                            (((0,), (0,)), ((), ())),
                            preferred_element_type=jnp.float32)
    h = jnp.maximum(h + b1[...], 0.0).astype(jnp.bfloat16)
    h = lax.dot_general(h, w2[...], (((1,), (0,)), ((), ())),
                        preferred_element_type=jnp.float32)
    h = jnp.maximum(h + b2[...], 0.0)
    z = jnp.sum(h * wfct[...], axis=1, keepdims=True) + scal[0, 0]
    sig = 1.0 / (1.0 + jnp.exp(-z))
    mn, mx = scal[0, 1], scal[0, 2]
    out[...] = sig * (mx - mn) + mn


def _mlp(xu, xn, idx4, W1, spad, b1, W2, b2, WfcT, scal, off, nrows):
    grid = (nrows // BT,)
    emb_spec = pl.BlockSpec((BT, NF), lambda i: (i, 0))
    idx_spec = pl.BlockSpec((4, BT), lambda i, off=off: (0, i + off))
    full = lambda shape: pl.BlockSpec(shape, lambda i: (0,) * len(shape))
    return pl.pallas_call(
        _mlp_body,
        grid=grid,
        in_specs=[
            emb_spec, emb_spec,
            idx_spec,
            full((6 * NF, H1)),
            full((NS_PAD, 4 * NF)),
            full((1, H1)),
            full((H1, H2)),
            full((1, H2)),
            full((1, H2)),
            pl.BlockSpec(memory_space=pltpu.SMEM),
        ],
        out_specs=pl.BlockSpec((BT, 1), lambda i: (i, 0)),
        out_shape=jax.ShapeDtypeStruct((nrows, 1), jnp.float32),
        scratch_shapes=[pltpu.VMEM((NS_PAD, H1), jnp.bfloat16)],
        compiler_params=pltpu.CompilerParams(
            dimension_semantics=("arbitrary",)),
    )(xu, xn, idx4, W1, spad, b1, W2, b2, WfcT, scal)


def kernel(users, news, gender, age, occupation, address,
           user_table, news_table, gender_table, age_table,
           occupation_table, address_table,
           W1, b1, W2, b2, Wfc, bfc,
           min_rating=0.5, max_rating=5.0):
    nrows = B // NCHUNK
    gather = _make_gather2(nrows)
    ui = users.astype(jnp.int32)
    ni = news.astype(jnp.int32)
    chunks = [gather(user_table, news_table,
                     ui[c * nrows:(c + 1) * nrows],
                     ni[c * nrows:(c + 1) * nrows])
              for c in range(NCHUNK)]
    spad = jnp.zeros((NS_PAD, 4 * NF), jnp.float32)
    spad = spad.at[0:2, 0:NF].set(gender_table)
    spad = spad.at[2:7, NF:2 * NF].set(age_table)
    spad = spad.at[7:17, 2 * NF:3 * NF].set(occupation_table)
    spad = spad.at[17:23, 3 * NF:4 * NF].set(address_table)
    scal = jnp.stack([
        jnp.asarray(bfc, jnp.float32).reshape(()),
        jnp.asarray(min_rating, jnp.float32).reshape(()),
        jnp.asarray(max_rating, jnp.float32).reshape(()),
    ]).reshape(1, 3)
    idx4 = jnp.stack([gender, age, occupation, address]).astype(jnp.int32)
    w1b = W1.astype(jnp.bfloat16)
    spadb = spad.astype(jnp.bfloat16)
    w2b = W2.astype(jnp.bfloat16)
    b1r = b1.reshape(1, H1)
    b2r = b2.reshape(1, H2)
    wfr = Wfc.reshape(1, H2)
    outs = [_mlp(xu, xn, idx4, w1b, spadb, b1r, w2b, b2r, wfr, scal,
                 c * (nrows // BT), nrows)
            for c, (xu, xn) in enumerate(chunks)]
    return jnp.concatenate(outs, axis=0)


# speedup vs baseline: 4.4222x; 1.1514x over previous
import functools

import jax
import jax.numpy as jnp
from jax import lax
from jax.experimental import pallas as pl
from jax.experimental.pallas import tpu as pltpu
from jax.experimental.pallas import tpu_sc as plsc

B = 4096
NF = 128
H1 = 1024
H2 = 512
BT = 2048
NS_PAD = 32
NCHUNK = 1


@functools.lru_cache(maxsize=None)
def _make_gather2(n):
    info = plsc.get_sparse_core_info()
    nc, ns = info.num_cores, info.num_subcores
    nw = nc * ns
    bpw = n // nw

    mesh = plsc.VectorSubcoreMesh(core_axis_name="c", subcore_axis_name="s")

    @functools.partial(
        pl.kernel,
        mesh=mesh,
        out_type=[jax.ShapeDtypeStruct((n, NF), jnp.float32) for _ in range(2)],
        scratch_types=[
            pltpu.VMEM((bpw,), jnp.int32),
            pltpu.VMEM((bpw,), jnp.int32),
            pltpu.VMEM((bpw, NF), jnp.float32),
            pltpu.VMEM((bpw, NF), jnp.float32),
            pltpu.SemaphoreType.DMA,
            pltpu.SemaphoreType.DMA,
            pltpu.SemaphoreType.DMA,
        ],
    )
    def gather2(ut, nt, ui, ni, xu, xn,
                idx_u, idx_n, rows_u, rows_n, sem_u, sem_n, sem_w):
        wid = lax.axis_index("s") * nc + lax.axis_index("c")
        base = wid * bpw
        iu = pltpu.async_copy(ui.at[pl.ds(base, bpw)], idx_u, sem_u)
        inn = pltpu.async_copy(ni.at[pl.ds(base, bpw)], idx_n, sem_n)
        iu.wait()
        inn.wait()
        cu = pltpu.async_copy(ut.at[idx_u], rows_u, sem_u)
        cn = pltpu.async_copy(nt.at[idx_n], rows_n, sem_n)
        cu.wait()
        wu = pltpu.async_copy(rows_u, xu.at[pl.ds(base, bpw)], sem_w)
        cn.wait()
        wn = pltpu.async_copy(rows_n, xn.at[pl.ds(base, bpw)], sem_w)
        wu.wait()
        wn.wait()

    return gather2


def _mlp_body(xu, xn, idx4, w1, spad, b1, w2, b2, wfct, scal, out, p):
    @pl.when(pl.program_id(0) == 0)
    def _():
        p[...] = lax.dot_general(spad[...], w1[2 * NF:6 * NF, :],
                                 (((1,), (0,)), ((), ())),
                                 preferred_element_type=jnp.float32
                                 ).astype(jnp.bfloat16)

    x2 = jnp.concatenate([xu[...], xn[...]], axis=1).astype(jnp.bfloat16)
    h = lax.dot_general(x2, w1[0:2 * NF, :], (((1,), (0,)), ((), ())),
                        preferred_element_type=jnp.float32)
    rows = lax.broadcasted_iota(jnp.int32, (NS_PAD, BT), 0)
    oht = ((rows == idx4[0:1, :]) | (rows == (idx4[1:2, :] + 2)) |
           (rows == (idx4[2:3, :] + 7)) | (rows == (idx4[3:4, :] + 17)))
    h = h + lax.dot_general(oht.astype(jnp.bfloat16), p[...],
                            (((0,), (0,)), ((), ())),
                            preferred_element_type=jnp.float32)
    h = jnp.maximum(h + b1[...], 0.0).astype(jnp.bfloat16)
    h = lax.dot_general(h, w2[...], (((1,), (0,)), ((), ())),
                        preferred_element_type=jnp.float32)
    h = jnp.maximum(h + b2[...], 0.0)
    z = jnp.sum(h * wfct[...], axis=1, keepdims=True) + scal[0, 0]
    sig = 1.0 / (1.0 + jnp.exp(-z))
    mn, mx = scal[0, 1], scal[0, 2]
    out[...] = sig * (mx - mn) + mn


def _mlp(xu, xn, idx4, W1, spad, b1, W2, b2, WfcT, scal, off, nrows):
    grid = (nrows // BT,)
    emb_spec = pl.BlockSpec((BT, NF), lambda i: (i, 0))
    idx_spec = pl.BlockSpec((4, BT), lambda i, off=off: (0, i + off))
    full = lambda shape: pl.BlockSpec(shape, lambda i: (0,) * len(shape))
    return pl.pallas_call(
        _mlp_body,
        grid=grid,
        in_specs=[
            emb_spec, emb_spec,
            idx_spec,
            full((6 * NF, H1)),
            full((NS_PAD, 4 * NF)),
            full((1, H1)),
            full((H1, H2)),
            full((1, H2)),
            full((1, H2)),
            pl.BlockSpec(memory_space=pltpu.SMEM),
        ],
        out_specs=pl.BlockSpec((BT, 1), lambda i: (i, 0)),
        out_shape=jax.ShapeDtypeStruct((nrows, 1), jnp.float32),
        scratch_shapes=[pltpu.VMEM((NS_PAD, H1), jnp.bfloat16)],
        compiler_params=pltpu.CompilerParams(
            dimension_semantics=("arbitrary",)),
    )(xu, xn, idx4, W1, spad, b1, W2, b2, WfcT, scal)


def kernel(users, news, gender, age, occupation, address,
           user_table, news_table, gender_table, age_table,
           occupation_table, address_table,
           W1, b1, W2, b2, Wfc, bfc,
           min_rating=0.5, max_rating=5.0):
    nrows = B // NCHUNK
    gather = _make_gather2(nrows)
    ui = users.astype(jnp.int32)
    ni = news.astype(jnp.int32)
    chunks = [gather(user_table, news_table,
                     ui[c * nrows:(c + 1) * nrows],
                     ni[c * nrows:(c + 1) * nrows])
              for c in range(NCHUNK)]
    spad = jnp.zeros((NS_PAD, 4 * NF), jnp.float32)
    spad = spad.at[0:2, 0:NF].set(gender_table)
    spad = spad.at[2:7, NF:2 * NF].set(age_table)
    spad = spad.at[7:17, 2 * NF:3 * NF].set(occupation_table)
    spad = spad.at[17:23, 3 * NF:4 * NF].set(address_table)
    scal = jnp.stack([
        jnp.asarray(bfc, jnp.float32).reshape(()),
        jnp.asarray(min_rating, jnp.float32).reshape(()),
        jnp.asarray(max_rating, jnp.float32).reshape(()),
    ]).reshape(1, 3)
    idx4 = jnp.stack([gender, age, occupation, address]).astype(jnp.int32)
    w1b = W1.astype(jnp.bfloat16)
    spadb = spad.astype(jnp.bfloat16)
    w2b = W2.astype(jnp.bfloat16)
    b1r = b1.reshape(1, H1)
    b2r = b2.reshape(1, H2)
    wfr = Wfc.reshape(1, H2)
    outs = [_mlp(xu, xn, idx4, w1b, spadb, b1r, w2b, b2r, wfr, scal,
                 c * (nrows // BT), nrows)
            for c, (xu, xn) in enumerate(chunks)]
    return jnp.concatenate(outs, axis=0)


# speedup vs baseline: 4.7970x; 1.0848x over previous
import functools

import jax
import jax.numpy as jnp
from jax import lax
from jax.experimental import pallas as pl
from jax.experimental.pallas import tpu as pltpu
from jax.experimental.pallas import tpu_sc as plsc

B = 4096
NF = 128
H1 = 1024
H2 = 512
BT = 2048
NS_PAD = 32
NCHUNK = 1


@functools.lru_cache(maxsize=None)
def _make_gather2(n):
    info = plsc.get_sparse_core_info()
    nc, ns = info.num_cores, info.num_subcores
    nw = nc * ns
    bpw = n // nw

    mesh = plsc.VectorSubcoreMesh(core_axis_name="c", subcore_axis_name="s")

    @functools.partial(
        pl.kernel,
        mesh=mesh,
        out_type=[jax.ShapeDtypeStruct((n, NF), jnp.float32) for _ in range(2)],
        scratch_types=[
            pltpu.VMEM((bpw,), jnp.int32),
            pltpu.VMEM((bpw,), jnp.int32),
            pltpu.VMEM((bpw, NF), jnp.float32),
            pltpu.VMEM((bpw, NF), jnp.float32),
            pltpu.SemaphoreType.DMA,
            pltpu.SemaphoreType.DMA,
            pltpu.SemaphoreType.DMA,
        ],
    )
    def gather2(ut, nt, ui, ni, xu, xn,
                idx_u, idx_n, rows_u, rows_n, sem_u, sem_n, sem_w):
        wid = lax.axis_index("s") * nc + lax.axis_index("c")
        base = wid * bpw
        iu = pltpu.async_copy(ui.at[pl.ds(base, bpw)], idx_u, sem_u)
        inn = pltpu.async_copy(ni.at[pl.ds(base, bpw)], idx_n, sem_n)
        iu.wait()
        inn.wait()
        cu = pltpu.async_copy(ut.at[idx_u], rows_u, sem_u)
        cn = pltpu.async_copy(nt.at[idx_n], rows_n, sem_n)
        cu.wait()
        wu = pltpu.async_copy(rows_u, xu.at[pl.ds(base, bpw)], sem_w)
        cn.wait()
        wn = pltpu.async_copy(rows_n, xn.at[pl.ds(base, bpw)], sem_w)
        wu.wait()
        wn.wait()

    return gather2


def _mlp_body(xu, xn, idx4, w1, spad, b1, w2, b2, wfct, scal, out, p):
    @pl.when(pl.program_id(0) == 0)
    def _():
        p[...] = lax.dot_general(spad[...], w1[2 * NF:6 * NF, :],
                                 (((1,), (0,)), ((), ())),
                                 preferred_element_type=jnp.float32
                                 ).astype(jnp.bfloat16)

    x2 = jnp.concatenate([xu[...], xn[...]], axis=1).astype(jnp.bfloat16)
    h = lax.dot_general(x2, w1[0:2 * NF, :], (((1,), (0,)), ((), ())),
                        preferred_element_type=jnp.float32)
    rows = lax.broadcasted_iota(jnp.int32, (NS_PAD, BT), 0)
    oht = ((rows == idx4[0:1, :]) | (rows == (idx4[1:2, :] + 2)) |
           (rows == (idx4[2:3, :] + 7)) | (rows == (idx4[3:4, :] + 17)))
    h = h + lax.dot_general(oht.astype(jnp.bfloat16), p[...],
                            (((0,), (0,)), ((), ())),
                            preferred_element_type=jnp.float32)
    h = jnp.maximum(h + b1[...], 0.0).astype(jnp.bfloat16)
    h = lax.dot_general(h, w2[...], (((1,), (0,)), ((), ())),
                        preferred_element_type=jnp.float32)
    h = jnp.maximum(h + b2[...], 0.0)
    z = lax.dot_general(wfct[...], h, (((1,), (1,)), ((), ())),
                        preferred_element_type=jnp.float32) + scal[0, 0]
    sig = 1.0 / (1.0 + jnp.exp(-z))
    mn, mx = scal[0, 1], scal[0, 2]
    out[...] = jnp.reshape(sig * (mx - mn) + mn, (BT // 128, 128))


def _mlp(xu, xn, idx4, W1, spad, b1, W2, b2, WfcT, scal, off, nrows):
    grid = (nrows // BT,)
    emb_spec = pl.BlockSpec((BT, NF), lambda i: (i, 0))
    idx_spec = pl.BlockSpec((4, BT), lambda i, off=off: (0, i + off))
    full = lambda shape: pl.BlockSpec(shape, lambda i: (0,) * len(shape))
    return pl.pallas_call(
        _mlp_body,
        grid=grid,
        in_specs=[
            emb_spec, emb_spec,
            idx_spec,
            full((6 * NF, H1)),
            full((NS_PAD, 4 * NF)),
            full((1, H1)),
            full((H1, H2)),
            full((1, H2)),
            full((1, H2)),
            pl.BlockSpec(memory_space=pltpu.SMEM),
        ],
        out_specs=pl.BlockSpec((BT // 128, 128), lambda i: (i, 0)),
        out_shape=jax.ShapeDtypeStruct((nrows // 128, 128), jnp.float32),
        scratch_shapes=[pltpu.VMEM((NS_PAD, H1), jnp.bfloat16)],
        compiler_params=pltpu.CompilerParams(
            dimension_semantics=("arbitrary",)),
    )(xu, xn, idx4, W1, spad, b1, W2, b2, WfcT, scal)


def kernel(users, news, gender, age, occupation, address,
           user_table, news_table, gender_table, age_table,
           occupation_table, address_table,
           W1, b1, W2, b2, Wfc, bfc,
           min_rating=0.5, max_rating=5.0):
    nrows = B // NCHUNK
    gather = _make_gather2(nrows)
    ui = users.astype(jnp.int32)
    ni = news.astype(jnp.int32)
    chunks = [gather(user_table, news_table,
                     ui[c * nrows:(c + 1) * nrows],
                     ni[c * nrows:(c + 1) * nrows])
              for c in range(NCHUNK)]
    spad = jnp.zeros((NS_PAD, 4 * NF), jnp.float32)
    spad = spad.at[0:2, 0:NF].set(gender_table)
    spad = spad.at[2:7, NF:2 * NF].set(age_table)
    spad = spad.at[7:17, 2 * NF:3 * NF].set(occupation_table)
    spad = spad.at[17:23, 3 * NF:4 * NF].set(address_table)
    scal = jnp.stack([
        jnp.asarray(bfc, jnp.float32).reshape(()),
        jnp.asarray(min_rating, jnp.float32).reshape(()),
        jnp.asarray(max_rating, jnp.float32).reshape(()),
    ]).reshape(1, 3)
    idx4 = jnp.stack([gender, age, occupation, address]).astype(jnp.int32)
    w1b = W1.astype(jnp.bfloat16)
    spadb = spad.astype(jnp.bfloat16)
    w2b = W2.astype(jnp.bfloat16)
    b1r = b1.reshape(1, H1)
    b2r = b2.reshape(1, H2)
    wfr = Wfc.reshape(1, H2)
    outs = [_mlp(xu, xn, idx4, w1b, spadb, b1r, w2b, b2r, wfr, scal,
                 c * (nrows // BT), nrows)
            for c, (xu, xn) in enumerate(chunks)]
    return jnp.concatenate(outs, axis=0).reshape(B, 1)
